# baseline (speedup 1.0000x reference)
import functools

import jax
import jax.numpy as jnp
from jax import lax
from jax.experimental import pallas as pl
from jax.experimental.pallas import tpu as pltpu
from jax.experimental.pallas import tpu_sc as plsc

B = 16384
D = 64
DH = 128
H = 256
N = 1000000

BLK = 65536
RPB = BLK // 128
NB = (N + BLK - 1) // BLK
RTOT = NB * RPB

NC = 2
NS = 16
L = 16
NW = NC * NS
BPW = B // NW
CHUNK = 128
NCH = BPW // CHUNK


def _collapse_body(w_hid_ref, b_hid_ref, w_out_ref, b_out_ref, w_ref, c_ref):
    w = lax.dot_general(w_hid_ref[...], w_out_ref[...],
                        (((1,), (0,)), ((), ())),
                        preferred_element_type=jnp.float32)
    w_ref[...] = w.reshape(1, DH)
    c = lax.dot_general(b_hid_ref[...], w_out_ref[...],
                        (((1,), (0,)), ((), ())),
                        preferred_element_type=jnp.float32)
    c_ref[...] = c + b_out_ref[...]


_collapse = pl.pallas_call(
    _collapse_body,
    out_shape=[
        jax.ShapeDtypeStruct((1, DH), jnp.float32),
        jax.ShapeDtypeStruct((1, 1), jnp.float32),
    ],
)


def _matvec_body(w_ref, x_ref, o_ref):
    r = lax.dot_general(w_ref[...], x_ref[...],
                        (((1,), (0,)), ((), ())),
                        preferred_element_type=jnp.float32)
    o_ref[...] = r.reshape(RPB, 128)


_matvec = pl.pallas_call(
    _matvec_body,
    grid=(NB,),
    in_specs=[
        pl.BlockSpec((1, D), lambda k: (0, 0)),
        pl.BlockSpec((D, BLK), lambda k: (0, k)),
    ],
    out_specs=pl.BlockSpec((RPB, 128), lambda k: (k, 0)),
    out_shape=jax.ShapeDtypeStruct((RTOT, 128), jnp.float32),
)


_sc_mesh = plsc.VectorSubcoreMesh(core_axis_name="c", subcore_axis_name="s")


@functools.partial(
    pl.kernel,
    out_type=jax.ShapeDtypeStruct((B,), jnp.float32),
    mesh=_sc_mesh,
    compiler_params=pltpu.CompilerParams(needs_layout_passes=False,
                                         use_tc_tiling_on_sc=False),
    scratch_types=[
        pltpu.VMEM((NCH, CHUNK), jnp.int32),
        pltpu.VMEM((BPW,), jnp.int32),
        pltpu.VMEM((2, CHUNK, 128), jnp.float32),
        pltpu.VMEM((BPW,), jnp.float32),
        pltpu.SemaphoreType.DMA,
        pltpu.SemaphoreType.DMA,
    ],
)
def _sc_gather(rows_hbm, lanes_hbm, dots_hbm, out_hbm,
               idx_v, lane_v, buf, out_v, sem_i, sem_g):
    wid = lax.axis_index("s") * NC + lax.axis_index("c")
    base = wid * BPW
    pltpu.sync_copy(rows_hbm.at[wid], idx_v)
    pltpu.sync_copy(lanes_hbm.at[pl.ds(base, BPW)], lane_v)

    copies = [None, None]

    def issue(j):
        copies[j % 2] = pltpu.async_copy(
            dots_hbm.at[idx_v.at[j]], buf.at[j % 2], sem_g)

    issue(0)
    for j in range(NCH):
        if j + 1 < NCH:
            issue(j + 1)
        copies[j % 2].wait()
        for blk in range(CHUNK // L):
            r0 = j * CHUNK + blk * L
            rid = blk * L + lax.iota(jnp.int32, L)
            lv = lane_v[pl.ds(r0, L)]
            out_v[pl.ds(r0, L)] = plsc.load_gather(buf.at[j % 2], [rid, lv])

    pltpu.sync_copy(out_v, out_hbm.at[pl.ds(base, BPW)])


def kernel(user, item, W_user, W_item, W_hid, b_hid, W_out, b_out):
    w2d, c2d = _collapse(W_hid, b_hid.reshape(1, H), W_out, b_out.reshape(1, 1))
    wu = w2d[:, :D]
    wi = w2d[:, D:]

    dots_u = _matvec(wu, W_user.T)
    dots_i = _matvec(wi, W_item.T)

    u = user.astype(jnp.int32)
    i = item.astype(jnp.int32)
    rows_u = lax.shift_right_logical(u, 7).reshape(NW, NCH, CHUNK)
    rows_i = lax.shift_right_logical(i, 7).reshape(NW, NCH, CHUNK)
    lanes_u = jnp.bitwise_and(u, 127)
    lanes_i = jnp.bitwise_and(i, 127)

    su = _sc_gather(rows_u, lanes_u, dots_u)
    si = _sc_gather(rows_i, lanes_i, dots_i)

    out = su + si + c2d[0, 0]
    return out.reshape(B, 1)


# speedup vs baseline: 1.0125x; 1.0125x over previous
import functools

import jax
import jax.numpy as jnp
from jax import lax
from jax.experimental import pallas as pl
from jax.experimental.pallas import tpu as pltpu
from jax.experimental.pallas import tpu_sc as plsc

B = 16384
D = 64
DH = 128
H = 256
N = 1000000

BLK = 32768
RPB = BLK // 128
NB = (N + BLK - 1) // BLK
RTOT = NB * RPB

NC = 2
NS = 16
L = 16
NW = NC * NS
BPW = B // NW
CHUNK = 128
NCH = BPW // CHUNK


def _collapse_body(w_hid_ref, b_hid_ref, w_out_ref, b_out_ref, w_ref, c_ref):
    w = lax.dot_general(w_hid_ref[...], w_out_ref[...],
                        (((1,), (0,)), ((), ())),
                        preferred_element_type=jnp.float32)
    w_ref[...] = w.reshape(1, DH)
    c = lax.dot_general(b_hid_ref[...], w_out_ref[...],
                        (((1,), (0,)), ((), ())),
                        preferred_element_type=jnp.float32)
    c_ref[...] = c + b_out_ref[...]


_collapse = pl.pallas_call(
    _collapse_body,
    out_shape=[
        jax.ShapeDtypeStruct((1, DH), jnp.float32),
        jax.ShapeDtypeStruct((1, 1), jnp.float32),
    ],
)


def _matvec_body(w_ref, x_ref, o_ref):
    r = lax.dot_general(w_ref[...], x_ref[...],
                        (((1,), (0,)), ((), ())),
                        preferred_element_type=jnp.float32)
    o_ref[...] = r.reshape(RPB, 128)


_matvec = pl.pallas_call(
    _matvec_body,
    grid=(NB,),
    in_specs=[
        pl.BlockSpec((1, D), lambda k: (0, 0)),
        pl.BlockSpec((D, BLK), lambda k: (0, k)),
    ],
    out_specs=pl.BlockSpec((RPB, 128), lambda k: (k, 0)),
    out_shape=jax.ShapeDtypeStruct((RTOT, 128), jnp.float32),
)


_sc_mesh = plsc.VectorSubcoreMesh(core_axis_name="c", subcore_axis_name="s")


@functools.partial(
    pl.kernel,
    out_type=jax.ShapeDtypeStruct((B,), jnp.float32),
    mesh=_sc_mesh,
    compiler_params=pltpu.CompilerParams(needs_layout_passes=False,
                                         use_tc_tiling_on_sc=False),
    scratch_types=[
        pltpu.VMEM((NCH, CHUNK), jnp.int32),
        pltpu.VMEM((BPW,), jnp.int32),
        pltpu.VMEM((2, CHUNK, 128), jnp.float32),
        pltpu.VMEM((BPW,), jnp.float32),
        pltpu.SemaphoreType.DMA,
        pltpu.SemaphoreType.DMA,
    ],
)
def _sc_gather(rows_hbm, lanes_hbm, dots_hbm, out_hbm,
               idx_v, lane_v, buf, out_v, sem_i, sem_g):
    wid = lax.axis_index("s") * NC + lax.axis_index("c")
    base = wid * BPW
    pltpu.sync_copy(rows_hbm.at[wid], idx_v)
    pltpu.sync_copy(lanes_hbm.at[pl.ds(base, BPW)], lane_v)

    copies = [None, None]

    def issue(j):
        copies[j % 2] = pltpu.async_copy(
            dots_hbm.at[idx_v.at[j]], buf.at[j % 2], sem_g)

    issue(0)
    for j in range(NCH):
        if j + 1 < NCH:
            issue(j + 1)
        copies[j % 2].wait()
        for blk in range(CHUNK // L):
            r0 = j * CHUNK + blk * L
            rid = blk * L + lax.iota(jnp.int32, L)
            lv = lane_v[pl.ds(r0, L)]
            out_v[pl.ds(r0, L)] = plsc.load_gather(buf.at[j % 2], [rid, lv])

    pltpu.sync_copy(out_v, out_hbm.at[pl.ds(base, BPW)])


def kernel(user, item, W_user, W_item, W_hid, b_hid, W_out, b_out):
    w2d, c2d = _collapse(W_hid, b_hid.reshape(1, H), W_out, b_out.reshape(1, 1))
    wu = w2d[:, :D]
    wi = w2d[:, D:]

    dots_u = _matvec(wu, W_user.T)
    dots_i = _matvec(wi, W_item.T)

    u = user.astype(jnp.int32)
    i = item.astype(jnp.int32)
    rows_u = lax.shift_right_logical(u, 7).reshape(NW, NCH, CHUNK)
    rows_i = lax.shift_right_logical(i, 7).reshape(NW, NCH, CHUNK)
    lanes_u = jnp.bitwise_and(u, 127)
    lanes_i = jnp.bitwise_and(i, 127)

    su = _sc_gather(rows_u, lanes_u, dots_u)
    si = _sc_gather(rows_i, lanes_i, dots_i)

    out = su + si + c2d[0, 0]
    return out.reshape(B, 1)
